# Initial kernel scaffold; baseline (speedup 1.0000x reference)
import jax
import jax.numpy as jnp
from jax.experimental import pallas as pl


def kernel(x, token_emb, pos_emb):
    raise NotImplementedError("write your pallas kernel here")


# speedup vs baseline: 1.4242x; 1.4242x over previous
import functools

import jax
import jax.numpy as jnp
from jax import lax
from jax.experimental import pallas as pl
from jax.experimental.pallas import tpu as pltpu
from jax.experimental.pallas import tpu_sc as plsc

MAXLEN = 200
EMBED = 32
LANES = 16
NWORKERS = 32
GATHER_ROWS = 100
GATHERS_PER_CHUNK = 16
CHUNK = GATHER_ROWS * GATHERS_PER_CHUNK


@functools.cache
def _make_kernel(n_rows: int):
    rows_per_w = n_rows // NWORKERS
    n_chunks = rows_per_w // CHUNK
    idx_rows_per_w = rows_per_w // GATHER_ROWS
    cycles = CHUNK // MAXLEN

    mesh = plsc.VectorSubcoreMesh(core_axis_name="c", subcore_axis_name="s")

    @functools.partial(
        pl.kernel,
        mesh=mesh,
        compiler_params=pltpu.CompilerParams(use_tc_tiling_on_sc=False),
        out_type=jax.ShapeDtypeStruct((n_rows, EMBED), jnp.float32),
        scratch_types=[
            pltpu.VMEM((GATHERS_PER_CHUNK, GATHER_ROWS), jnp.int32),
            pltpu.VMEM((CHUNK, EMBED), jnp.float32),
            pltpu.VMEM((MAXLEN, EMBED), jnp.float32),
            pltpu.SemaphoreType.DMA,
        ],
    )
    def emb_kernel(x_hbm, tok_hbm, pos_hbm, out_hbm, idx_v, rows_v, pos_v, sem):
        wid = lax.axis_index("s") * 2 + lax.axis_index("c")
        pltpu.sync_copy(pos_hbm, pos_v)

        def chunk_body(g, carry):
            r0 = wid * idx_rows_per_w + g * GATHERS_PER_CHUNK
            pltpu.sync_copy(x_hbm.at[pl.ds(r0, GATHERS_PER_CHUNK)], idx_v)
            copies = [
                pltpu.async_copy(
                    tok_hbm.at[idx_v.at[j]],
                    rows_v.at[pl.ds(j * GATHER_ROWS, GATHER_ROWS)],
                    sem,
                )
                for j in range(GATHERS_PER_CHUNK)
            ]
            for c in copies:
                c.wait()

            def t_body(t, carry2):
                p0 = pos_v[t, pl.ds(0, LANES)]
                p1 = pos_v[t, pl.ds(LANES, LANES)]
                for cyc in range(cycles):
                    r = cyc * MAXLEN + t
                    rows_v[r, pl.ds(0, LANES)] = rows_v[r, pl.ds(0, LANES)] + p0
                    rows_v[r, pl.ds(LANES, LANES)] = (
                        rows_v[r, pl.ds(LANES, LANES)] + p1
                    )
                return carry2

            lax.fori_loop(0, MAXLEN, t_body, 0)

            base = wid * rows_per_w + g * CHUNK
            pltpu.sync_copy(rows_v, out_hbm.at[pl.ds(base, CHUNK)])
            return carry

        lax.fori_loop(0, n_chunks, chunk_body, 0)

    return emb_kernel


def kernel(x, token_emb, pos_emb):
    batch, maxlen = x.shape
    n_rows = batch * maxlen
    x2 = x.reshape(n_rows // GATHER_ROWS, GATHER_ROWS).astype(jnp.int32)
    out = _make_kernel(n_rows)(x2, token_emb, pos_emb)
    return out.reshape(batch, maxlen, EMBED)


# speedup vs baseline: 1.8879x; 1.3256x over previous
import functools

import jax
import jax.numpy as jnp
from jax import lax
from jax.experimental import pallas as pl
from jax.experimental.pallas import tpu as pltpu
from jax.experimental.pallas import tpu_sc as plsc

MAXLEN = 200
EMBED = 32
LANES = 16
NWORKERS = 32
GATHER_ROWS = 100
GATHERS_PER_CHUNK = 16
CHUNK = GATHER_ROWS * GATHERS_PER_CHUNK

VOCAB = 1000000
LIN_BLOCK_V = 2048


def _linearize_table(tok_t):
    grid = (VOCAB + LIN_BLOCK_V - 1) // LIN_BLOCK_V
    rows_out = LIN_BLOCK_V // 4

    def body(in_ref, out_ref):
        y = in_ref[...].T.reshape(rows_out, 4, EMBED)
        out_ref[...] = jnp.concatenate(
            [y[:, 0, :], y[:, 1, :], y[:, 2, :], y[:, 3, :]], axis=1
        )

    return pl.pallas_call(
        body,
        grid=(grid,),
        in_specs=[pl.BlockSpec((EMBED, LIN_BLOCK_V), lambda i: (0, i))],
        out_specs=pl.BlockSpec((rows_out, 128), lambda i: (i, 0)),
        out_shape=jax.ShapeDtypeStruct((VOCAB * EMBED // 128, 128), jnp.float32),
    )(tok_t)


OUT_BLOCK_B = 128


def _transpose_out(rows128, batch):
    per_b128 = MAXLEN * EMBED // 128
    grid = batch // OUT_BLOCK_B
    rows_in = OUT_BLOCK_B * per_b128

    def body(in_ref, out_ref):
        blk = in_ref[...].reshape(OUT_BLOCK_B, per_b128, 128)
        y = jnp.transpose(blk, (1, 2, 0))
        out_ref[...] = y.reshape(MAXLEN, EMBED, OUT_BLOCK_B)

    return pl.pallas_call(
        body,
        grid=(grid,),
        in_specs=[pl.BlockSpec((rows_in, 128), lambda i: (i, 0))],
        out_specs=pl.BlockSpec((MAXLEN, EMBED, OUT_BLOCK_B), lambda i: (0, 0, i)),
        out_shape=jax.ShapeDtypeStruct((MAXLEN, EMBED, batch), jnp.float32),
    )(rows128)


@functools.cache
def _make_gather_kernel(n_rows: int):
    rows_per_w = n_rows // NWORKERS
    n_chunks = rows_per_w // CHUNK
    idx_rows_per_w = rows_per_w // GATHER_ROWS
    cycles = CHUNK // MAXLEN

    mesh = plsc.VectorSubcoreMesh(core_axis_name="c", subcore_axis_name="s")

    @functools.partial(
        pl.kernel,
        mesh=mesh,
        compiler_params=pltpu.CompilerParams(use_tc_tiling_on_sc=False),
        out_type=jax.ShapeDtypeStruct((n_rows, EMBED), jnp.float32),
        scratch_types=[
            pltpu.VMEM((GATHERS_PER_CHUNK, GATHER_ROWS), jnp.int32),
            pltpu.VMEM((CHUNK, EMBED), jnp.float32),
            pltpu.VMEM((MAXLEN, EMBED), jnp.float32),
            pltpu.SemaphoreType.DMA,
        ],
    )
    def emb_kernel(x_hbm, tok_hbm, pos_hbm, out_hbm, idx_v, rows_v, pos_v, sem):
        wid = lax.axis_index("s") * 2 + lax.axis_index("c")
        pltpu.sync_copy(pos_hbm, pos_v)

        def chunk_body(g, carry):
            r0 = wid * idx_rows_per_w + g * GATHERS_PER_CHUNK
            pltpu.sync_copy(x_hbm.at[pl.ds(r0, GATHERS_PER_CHUNK)], idx_v)
            copies = [
                pltpu.async_copy(
                    tok_hbm.at[idx_v.at[j]],
                    rows_v.at[pl.ds(j * GATHER_ROWS, GATHER_ROWS)],
                    sem,
                )
                for j in range(GATHERS_PER_CHUNK)
            ]
            for c in copies:
                c.wait()

            def t_body(t, carry2):
                p0 = pos_v[t, pl.ds(0, LANES)]
                p1 = pos_v[t, pl.ds(LANES, LANES)]
                for cyc in range(cycles):
                    r = cyc * MAXLEN + t
                    rows_v[r, pl.ds(0, LANES)] = rows_v[r, pl.ds(0, LANES)] + p0
                    rows_v[r, pl.ds(LANES, LANES)] = (
                        rows_v[r, pl.ds(LANES, LANES)] + p1
                    )
                return carry2

            lax.fori_loop(0, MAXLEN, t_body, 0)

            base = wid * rows_per_w + g * CHUNK
            pltpu.sync_copy(rows_v, out_hbm.at[pl.ds(base, CHUNK)])
            return carry

        lax.fori_loop(0, n_chunks, chunk_body, 0)

    return emb_kernel


def kernel(x, token_emb, pos_emb):
    batch, maxlen = x.shape
    n_rows = batch * maxlen
    x2 = x.reshape(n_rows // GATHER_ROWS, GATHER_ROWS).astype(jnp.int32)
    tok_lin = _linearize_table(token_emb.T).reshape(VOCAB, EMBED)
    rows = _make_gather_kernel(n_rows)(x2, tok_lin, pos_emb)
    out_tdb = _transpose_out(rows.reshape(n_rows * EMBED // 128, 128), batch)
    return jnp.transpose(out_tdb, (2, 0, 1))


# speedup vs baseline: 1.9967x; 1.0576x over previous
import functools

import jax
import jax.numpy as jnp
from jax import lax
from jax.experimental import pallas as pl
from jax.experimental.pallas import tpu as pltpu
from jax.experimental.pallas import tpu_sc as plsc

MAXLEN = 200
EMBED = 32
LANES = 16
NWORKERS = 32
GATHER_ROWS = 100
GATHERS_PER_CHUNK = 16
CHUNK = GATHER_ROWS * GATHERS_PER_CHUNK

VOCAB = 1000000
LIN_BLOCK_V = 2048


def _linearize_table(tok_t):
    grid = (VOCAB + LIN_BLOCK_V - 1) // LIN_BLOCK_V
    rows_out = LIN_BLOCK_V // 4

    def body(in_ref, out_ref):
        x = in_ref[...]
        z = jnp.concatenate([x, x, x, x], axis=0)
        zt = z.T.reshape(rows_out, 4, 128)
        lane = lax.broadcasted_iota(jnp.int32, (rows_out, 128), 1)
        q = lane // EMBED
        out = zt[:, 0, :]
        for k in range(1, 4):
            out = jnp.where(q == k, zt[:, k, :], out)
        out_ref[...] = out

    return pl.pallas_call(
        body,
        grid=(grid,),
        in_specs=[pl.BlockSpec((EMBED, LIN_BLOCK_V), lambda i: (0, i))],
        out_specs=pl.BlockSpec((rows_out, 128), lambda i: (i, 0)),
        out_shape=jax.ShapeDtypeStruct((VOCAB * EMBED // 128, 128), jnp.float32),
    )(tok_t)


OUT_BLOCK_B = 128


def _transpose_out(rows128, batch):
    per_b128 = MAXLEN * EMBED // 128
    grid = batch // OUT_BLOCK_B
    rows_in = OUT_BLOCK_B * per_b128

    def body(in_ref, out_ref):
        blk = in_ref[...].reshape(OUT_BLOCK_B, per_b128, 128)
        y = jnp.transpose(blk, (1, 2, 0))
        out_ref[...] = y.reshape(MAXLEN, EMBED, OUT_BLOCK_B)

    return pl.pallas_call(
        body,
        grid=(grid,),
        in_specs=[pl.BlockSpec((rows_in, 128), lambda i: (i, 0))],
        out_specs=pl.BlockSpec((MAXLEN, EMBED, OUT_BLOCK_B), lambda i: (0, 0, i)),
        out_shape=jax.ShapeDtypeStruct((MAXLEN, EMBED, batch), jnp.float32),
    )(rows128)


@functools.cache
def _make_gather_kernel(n_rows: int):
    rows_per_w = n_rows // NWORKERS
    n_chunks = rows_per_w // CHUNK
    idx_rows_per_w = rows_per_w // GATHER_ROWS
    cycles = CHUNK // MAXLEN

    mesh = plsc.VectorSubcoreMesh(core_axis_name="c", subcore_axis_name="s")

    @functools.partial(
        pl.kernel,
        mesh=mesh,
        compiler_params=pltpu.CompilerParams(use_tc_tiling_on_sc=False),
        out_type=jax.ShapeDtypeStruct((n_rows, EMBED), jnp.float32),
        scratch_types=[
            pltpu.VMEM((GATHERS_PER_CHUNK, GATHER_ROWS), jnp.int32),
            pltpu.VMEM((CHUNK, EMBED), jnp.float32),
            pltpu.VMEM((MAXLEN, EMBED), jnp.float32),
            pltpu.SemaphoreType.DMA,
        ],
    )
    def emb_kernel(x_hbm, tok_hbm, pos_hbm, out_hbm, idx_v, rows_v, pos_v, sem):
        wid = lax.axis_index("s") * 2 + lax.axis_index("c")
        pltpu.sync_copy(pos_hbm, pos_v)

        def chunk_body(g, carry):
            r0 = wid * idx_rows_per_w + g * GATHERS_PER_CHUNK
            pltpu.sync_copy(x_hbm.at[pl.ds(r0, GATHERS_PER_CHUNK)], idx_v)
            copies = [
                pltpu.async_copy(
                    tok_hbm.at[idx_v.at[j]],
                    rows_v.at[pl.ds(j * GATHER_ROWS, GATHER_ROWS)],
                    sem,
                )
                for j in range(GATHERS_PER_CHUNK)
            ]
            for c in copies:
                c.wait()

            def t_body(t, carry2):
                p0 = pos_v[t, pl.ds(0, LANES)]
                p1 = pos_v[t, pl.ds(LANES, LANES)]
                for cyc in range(cycles):
                    r = cyc * MAXLEN + t
                    rows_v[r, pl.ds(0, LANES)] = rows_v[r, pl.ds(0, LANES)] + p0
                    rows_v[r, pl.ds(LANES, LANES)] = (
                        rows_v[r, pl.ds(LANES, LANES)] + p1
                    )
                return carry2

            lax.fori_loop(0, MAXLEN, t_body, 0)

            base = wid * rows_per_w + g * CHUNK
            pltpu.sync_copy(rows_v, out_hbm.at[pl.ds(base, CHUNK)])
            return carry

        lax.fori_loop(0, n_chunks, chunk_body, 0)

    return emb_kernel


def kernel(x, token_emb, pos_emb):
    batch, maxlen = x.shape
    n_rows = batch * maxlen
    x2 = x.reshape(n_rows // GATHER_ROWS, GATHER_ROWS).astype(jnp.int32)
    tok_lin = _linearize_table(token_emb.T).reshape(VOCAB, EMBED)
    rows = _make_gather_kernel(n_rows)(x2, tok_lin, pos_emb)
    out_tdb = _transpose_out(rows.reshape(n_rows * EMBED // 128, 128), batch)
    return jnp.transpose(out_tdb, (2, 0, 1))
